# baseline (speedup 1.0000x reference)
import functools

import jax
import jax.numpy as jnp
from jax import lax
from jax.experimental import pallas as pl
from jax.experimental.pallas import tpu as pltpu
from jax.experimental.pallas import tpu_sc as plsc

_NC = 2
_NS = 16
_NW = _NC * _NS
_L = 16
_NBUF = 4

_PACK_BLK = 2048
_PACK_B = 248 * _PACK_BLK


def _tc_pack(table_t, vocab, embed):
    nblk = _PACK_B // _PACK_BLK

    last_blk = (vocab - 1) // _PACK_BLK

    def body(lo_ref, hi_ref, o_ref):
        o_ref[:, 0:embed] = jnp.transpose(lo_ref[...])
        o_ref[:, embed:2 * embed] = jnp.transpose(hi_ref[...])

    return pl.pallas_call(
        body,
        grid=(nblk,),
        in_specs=[
            pl.BlockSpec((embed, _PACK_BLK), lambda i: (0, i)),
            pl.BlockSpec(
                (embed, _PACK_BLK),
                lambda i: (0, jnp.minimum(i + nblk, last_blk)),
            ),
        ],
        out_specs=pl.BlockSpec((_PACK_BLK, 2 * embed), lambda i: (i, 0)),
        out_shape=jax.ShapeDtypeStruct((_PACK_B, 2 * embed), jnp.float32),
    )(table_t, table_t)


def _sc_pool(x_by_worker, packed, batch, hist, embed):
    bpw = batch // _NW
    assert bpw % 8 == 0 and bpw <= 128
    nvec = embed // _L
    width = 2 * embed
    hist_pad = hist + (-hist) % _NBUF
    mesh = plsc.VectorSubcoreMesh(core_axis_name="c", subcore_axis_name="s")

    @functools.partial(
        pl.kernel,
        mesh=mesh,
        compiler_params=pltpu.CompilerParams(
            use_tc_tiling_on_sc=False, needs_layout_passes=False
        ),
        out_type=jax.ShapeDtypeStruct((embed, batch), jnp.float32),
        scratch_types=(
            [pltpu.VMEM((bpw * hist,), jnp.int32)]
            + [pltpu.VMEM((hist_pad, bpw), jnp.int32)]
            + [pltpu.VMEM((hist_pad, bpw), jnp.int32)]
            + [pltpu.VMEM((bpw, width), jnp.float32) for _ in range(_NBUF)]
            + [pltpu.VMEM((embed, bpw), jnp.float32)]
            + [pltpu.SemaphoreType.DMA for _ in range(_NBUF)]
        ),
    )
    def pool(x_hbm, tab_hbm, out_hbm, idx_raw, idx_v, off_v, *rest):
        bufs = rest[:_NBUF]
        acc = rest[_NBUF]
        sems = rest[_NBUF + 1:]
        wid = lax.axis_index("s") * _NC + lax.axis_index("c")
        base = wid * bpw

        pltpu.sync_copy(x_hbm.at[wid], idx_raw)

        lanes = jnp.arange(_L, dtype=jnp.int32)
        def tbody(j, c):
            for g in range(bpw // _L):
                offs = (g * _L + lanes) * hist + j
                v = plsc.load_gather(idx_raw, [offs])
                ge = (v >= _PACK_B).astype(jnp.int32)
                sl = pl.ds(g * _L, _L)
                idx_v[j, sl] = v - ge * _PACK_B
                off_v[j, sl] = ge * embed
            return c
        lax.fori_loop(0, hist, tbody, 0, unroll=2)

        zi = jnp.zeros((_L,), jnp.int32)
        for j in range(hist, hist_pad):
            for g in range(bpw // _L):
                idx_v[j, pl.ds(g * _L, _L)] = zi

        def gcopy(j, b):
            return pltpu.make_async_copy(
                tab_hbm.at[idx_v.at[j]], bufs[b], sems[b]
            )

        zeros = jnp.zeros((_L,), jnp.float32)
        for j in range(_NBUF):
            gcopy(j, j).start()
        def zbody(e, c):
            for g in range(bpw // _L):
                acc[e, pl.ds(g * _L, _L)] = zeros
            return c
        lax.fori_loop(0, embed, zbody, 0, unroll=4)

        rows_g = [g * _L + lanes for g in range(bpw // _L)]

        def accum(j, buf):
            offs = [off_v[j, pl.ds(g * _L, _L)] for g in range(bpw // _L)]
            def body(e, c):
                for g in range(bpw // _L):
                    sl = pl.ds(g * _L, _L)
                    col = offs[g] + e
                    acc[e, sl] = acc[e, sl] + plsc.load_gather(
                        buf, [rows_g[g], col]
                    )
                return c
            lax.fori_loop(0, embed, body, 0, unroll=2)

        def ring(g, c):
            j0 = g * _NBUF
            for b in range(_NBUF):
                j = j0 + b
                gcopy(j, b).wait()
                accum(j, bufs[b])
                gcopy(j + _NBUF, b).start()
            return c
        lax.fori_loop(0, hist_pad // _NBUF - 1, ring, 0)

        for j in range(hist_pad - _NBUF, hist_pad):
            b = j % _NBUF
            gcopy(j, b).wait()
            if j < hist:
                accum(j, bufs[b])

        pltpu.sync_copy(acc, out_hbm.at[:, pl.ds(base, bpw)])

    return pool(x_by_worker, packed)


def _tc_mlp_t(pooled_t, w1t, b1c, w2t, b2c, batch, embed, ncls):
    blk = 512

    def body(p_ref, w1_ref, b1_ref, w2_ref, b2_ref, o_ref):
        h = jnp.dot(w1_ref[...], p_ref[...], preferred_element_type=jnp.float32)
        h = jnp.maximum(h + b1_ref[...], 0.0)
        o = jnp.dot(w2_ref[...], h, preferred_element_type=jnp.float32)
        o_ref[...] = o + b2_ref[...]

    return pl.pallas_call(
        body,
        grid=(batch // blk,),
        in_specs=[
            pl.BlockSpec((embed, blk), lambda i: (0, i)),
            pl.BlockSpec((embed, embed), lambda i: (0, 0)),
            pl.BlockSpec((embed, 1), lambda i: (0, 0)),
            pl.BlockSpec((ncls, embed), lambda i: (0, 0)),
            pl.BlockSpec((ncls, 1), lambda i: (0, 0)),
        ],
        out_specs=pl.BlockSpec((ncls, blk), lambda i: (0, i)),
        out_shape=jax.ShapeDtypeStruct((ncls, batch), jnp.float32),
    )(pooled_t, w1t, b1c, w2t, b2c)


def kernel(x, table, W1, b1, W2, b2):
    batch, hist = x.shape
    vocab, embed = table.shape
    ncls = W2.shape[1]

    packed = _tc_pack(table.T, vocab, embed)

    bpw = batch // _NW
    xw = x.astype(jnp.int32).reshape(_NW, bpw * hist)

    pooled_sum_t = _sc_pool(xw, packed, batch, hist, embed)

    w1t = W1.T * (1.0 / hist)
    out_t = _tc_mlp_t(
        pooled_sum_t,
        w1t,
        b1.reshape(embed, 1),
        W2.T,
        b2.reshape(ncls, 1),
        batch,
        embed,
        ncls,
    )
    return out_t.T


# speedup vs baseline: 1.5672x; 1.5672x over previous
import functools

import jax
import jax.numpy as jnp
from jax import lax
from jax.experimental import pallas as pl
from jax.experimental.pallas import tpu as pltpu
from jax.experimental.pallas import tpu_sc as plsc

_NC = 2
_NS = 16
_NW = _NC * _NS
_L = 16
_NBUF = 4

_PACK_BLK = 2048
_PACK_B = 248 * _PACK_BLK


def _tc_pack(table_t, vocab, embed):
    nblk = _PACK_B // _PACK_BLK

    last_blk = (vocab - 1) // _PACK_BLK

    def body(lo_ref, hi_ref, o_ref):
        @pl.when(pl.program_id(0) < nblk)
        def _():
            o_ref[:, 0:embed] = jnp.transpose(lo_ref[...])
            o_ref[:, embed:2 * embed] = jnp.transpose(hi_ref[...])

        @pl.when(pl.program_id(0) == nblk)
        def _():
            o_ref[...] = jnp.zeros_like(o_ref)

    return pl.pallas_call(
        body,
        grid=(nblk + 1,),
        in_specs=[
            pl.BlockSpec(
                (embed, _PACK_BLK),
                lambda i: (0, jnp.minimum(i, last_blk)),
            ),
            pl.BlockSpec(
                (embed, _PACK_BLK),
                lambda i: (0, jnp.minimum(i + nblk, last_blk)),
            ),
        ],
        out_specs=pl.BlockSpec((_PACK_BLK, 2 * embed), lambda i: (i, 0)),
        out_shape=jax.ShapeDtypeStruct(
            (_PACK_B + _PACK_BLK, 2 * embed), jnp.float32
        ),
    )(table_t, table_t)


def _sc_pool(x_by_worker, packed_flat, batch, hist, embed):
    bpw = batch // _NW
    assert bpw % 8 == 0 and bpw <= 128
    nvec = embed // _L
    hist_pad = hist + (-hist) % _NBUF
    mesh = plsc.VectorSubcoreMesh(core_axis_name="c", subcore_axis_name="s")

    @functools.partial(
        pl.kernel,
        mesh=mesh,
        compiler_params=pltpu.CompilerParams(
            use_tc_tiling_on_sc=False, needs_layout_passes=False
        ),
        out_type=jax.ShapeDtypeStruct((batch, embed), jnp.float32),
        scratch_types=(
            [pltpu.VMEM((bpw * hist,), jnp.int32)]
            + [pltpu.VMEM((hist_pad, bpw), jnp.int32)]
            + [pltpu.VMEM((bpw, embed), jnp.float32) for _ in range(_NBUF)]
            + [pltpu.VMEM((bpw, embed), jnp.float32)]
            + [pltpu.SemaphoreType.DMA for _ in range(_NBUF)]
        ),
    )
    def pool(x_hbm, tab_hbm, out_hbm, idx_raw, idx_v, *rest):
        bufs = rest[:_NBUF]
        acc = rest[_NBUF]
        sems = rest[_NBUF + 1:]
        wid = lax.axis_index("s") * _NC + lax.axis_index("c")
        base = wid * bpw

        pltpu.sync_copy(x_hbm.at[wid], idx_raw)

        lanes = jnp.arange(_L, dtype=jnp.int32)
        def tbody(j, c):
            for g in range(bpw // _L):
                offs = (g * _L + lanes) * hist + j
                v = plsc.load_gather(idx_raw, [offs])
                ge = (v >= _PACK_B).astype(jnp.int32)
                idx_v[j, pl.ds(g * _L, _L)] = 2 * v - ge * (2 * _PACK_B - 1)
            return c
        lax.fori_loop(0, hist, tbody, 0, unroll=2)

        zi = jnp.zeros((_L,), jnp.int32)
        for j in range(hist, hist_pad):
            for g in range(bpw // _L):
                idx_v[j, pl.ds(g * _L, _L)] = zi

        def gcopy(j, b):
            return pltpu.make_async_copy(
                tab_hbm.at[idx_v.at[j]], bufs[b], sems[b]
            )

        zeros = jnp.zeros((_L,), jnp.float32)
        for j in range(_NBUF):
            gcopy(j, j).start()
        def zbody(r, c):
            for p in range(nvec):
                acc[r, pl.ds(p * _L, _L)] = zeros
            return c
        lax.fori_loop(0, bpw, zbody, 0, unroll=8)

        def accum(buf):
            def body(r, c):
                for p in range(nvec):
                    sl = pl.ds(p * _L, _L)
                    acc[r, sl] = acc[r, sl] + buf[r, sl]
                return c
            lax.fori_loop(0, bpw, body, 0, unroll=4)

        def ring(g, c):
            j0 = g * _NBUF
            for b in range(_NBUF):
                j = j0 + b
                gcopy(j, b).wait()
                accum(bufs[b])
                gcopy(j + _NBUF, b).start()
            return c
        lax.fori_loop(0, hist_pad // _NBUF - 1, ring, 0)

        for j in range(hist_pad - _NBUF, hist_pad):
            b = j % _NBUF
            gcopy(j, b).wait()
            if j < hist:
                accum(bufs[b])

        pltpu.sync_copy(acc, out_hbm.at[pl.ds(base, bpw)])

    return pool(x_by_worker, packed_flat)


def _tc_mlp(pooled, w1, b1, w2, b2, batch, embed, ncls):
    blk = 512

    def body(p_ref, w1_ref, b1_ref, w2_ref, b2_ref, o_ref):
        h = jnp.dot(p_ref[...], w1_ref[...], preferred_element_type=jnp.float32)
        h = jnp.maximum(h + b1_ref[...], 0.0)
        o = jnp.dot(h, w2_ref[...], preferred_element_type=jnp.float32)
        o_ref[...] = o + b2_ref[...]

    return pl.pallas_call(
        body,
        grid=(batch // blk,),
        in_specs=[
            pl.BlockSpec((blk, embed), lambda i: (i, 0)),
            pl.BlockSpec((embed, embed), lambda i: (0, 0)),
            pl.BlockSpec((1, embed), lambda i: (0, 0)),
            pl.BlockSpec((embed, ncls), lambda i: (0, 0)),
            pl.BlockSpec((1, ncls), lambda i: (0, 0)),
        ],
        out_specs=pl.BlockSpec((blk, ncls), lambda i: (i, 0)),
        out_shape=jax.ShapeDtypeStruct((batch, ncls), jnp.float32),
    )(pooled, w1, b1, w2, b2)


def kernel(x, table, W1, b1, W2, b2):
    batch, hist = x.shape
    vocab, embed = table.shape
    ncls = W2.shape[1]

    packed = _tc_pack(table.T, vocab, embed)
    packed_flat = packed.reshape(-1, embed)

    bpw = batch // _NW
    xw = x.astype(jnp.int32).reshape(_NW, bpw * hist)

    pooled_sum = _sc_pool(xw, packed_flat, batch, hist, embed)

    w1s = W1 * (1.0 / hist)
    out = _tc_mlp(
        pooled_sum,
        w1s,
        b1.reshape(1, embed),
        W2,
        b2.reshape(1, ncls),
        batch,
        embed,
        ncls,
    )
    return out


# speedup vs baseline: 2.0269x; 1.2934x over previous
import functools

import jax
import jax.numpy as jnp
from jax import lax
from jax.experimental import pallas as pl
from jax.experimental.pallas import tpu as pltpu
from jax.experimental.pallas import tpu_sc as plsc

_NC = 2
_NS = 16
_NW = _NC * _NS
_L = 16
_NBUF = 4

_PACK_BLK = 2048
_PACK_B = 248 * _PACK_BLK


def _tc_pack(table_t, vocab, embed):
    nblk = _PACK_B // _PACK_BLK

    last_blk = (vocab - 1) // _PACK_BLK

    def body(lo_ref, hi_ref, o_ref):
        @pl.when(pl.program_id(0) < nblk)
        def _():
            o_ref[:, 0:embed] = jnp.transpose(lo_ref[...])
            o_ref[:, embed:2 * embed] = jnp.transpose(hi_ref[...])

        @pl.when(pl.program_id(0) == nblk)
        def _():
            o_ref[...] = jnp.zeros_like(o_ref)

    return pl.pallas_call(
        body,
        grid=(nblk + 1,),
        in_specs=[
            pl.BlockSpec(
                (embed, _PACK_BLK),
                lambda i: (0, jnp.minimum(i, last_blk)),
            ),
            pl.BlockSpec(
                (embed, _PACK_BLK),
                lambda i: (0, jnp.minimum(i + nblk, last_blk)),
            ),
        ],
        out_specs=pl.BlockSpec((_PACK_BLK, 2 * embed), lambda i: (i, 0)),
        out_shape=jax.ShapeDtypeStruct(
            (_PACK_B + _PACK_BLK, 2 * embed), jnp.float32
        ),
    )(table_t, table_t)


def _sc_pool(x_by_worker, packed_flat, batch, hist, embed):
    bpw = batch // _NW
    assert bpw % 8 == 0 and bpw <= 128
    nvec = embed // _L
    hist_pad = hist + (-hist) % _NBUF
    mesh = plsc.VectorSubcoreMesh(core_axis_name="c", subcore_axis_name="s")

    @functools.partial(
        pl.kernel,
        mesh=mesh,
        compiler_params=pltpu.CompilerParams(
            use_tc_tiling_on_sc=False, needs_layout_passes=False
        ),
        out_type=jax.ShapeDtypeStruct((batch, embed), jnp.float32),
        scratch_types=(
            [pltpu.VMEM((bpw * hist,), jnp.int32)]
            + [pltpu.VMEM((hist, bpw), jnp.int32)]
            + [pltpu.VMEM((bpw, embed), jnp.float32) for _ in range(_NBUF)]
            + [pltpu.VMEM((bpw, embed), jnp.float32)]
            + [pltpu.SemaphoreType.DMA for _ in range(_NBUF)]
        ),
    )
    def pool(x_hbm, tab_hbm, out_hbm, idx_raw, idx_v, *rest):
        bufs = rest[:_NBUF]
        acc = rest[_NBUF]
        sems = rest[_NBUF + 1:]
        wid = lax.axis_index("s") * _NC + lax.axis_index("c")
        base = wid * bpw

        pltpu.sync_copy(x_hbm.at[wid], idx_raw)

        lanes = jnp.arange(_L, dtype=jnp.int32)
        def tbody(j, c):
            for g in range(bpw // _L):
                offs = (g * _L + lanes) * hist + j
                v = plsc.load_gather(idx_raw, [offs])
                ge = (v >= _PACK_B).astype(jnp.int32)
                idx_v[j, pl.ds(g * _L, _L)] = 2 * v - ge * (2 * _PACK_B - 1)
            return c
        lax.fori_loop(0, hist, tbody, 0, unroll=2)

        def gcopy(j, b):
            return pltpu.make_async_copy(
                tab_hbm.at[idx_v.at[j]], bufs[b], sems[b]
            )

        zeros = jnp.zeros((_L,), jnp.float32)
        for j in range(_NBUF):
            gcopy(j, j).start()
        def zbody(r, c):
            for p in range(nvec):
                acc[r, pl.ds(p * _L, _L)] = zeros
            return c
        lax.fori_loop(0, bpw, zbody, 0, unroll=8)

        def accum(buf):
            def body(r, c):
                for p in range(nvec):
                    sl = pl.ds(p * _L, _L)
                    acc[r, sl] = acc[r, sl] + buf[r, sl]
                return c
            lax.fori_loop(0, bpw, body, 0, unroll=4)

        for j in range(hist):
            b = j % _NBUF
            gcopy(j, b).wait()
            accum(bufs[b])
            nj = j + _NBUF
            if nj < hist:
                gcopy(nj, b).start()

        pltpu.sync_copy(acc, out_hbm.at[pl.ds(base, bpw)])

    return pool(x_by_worker, packed_flat)


def _tc_mlp(pooled, w1, b1, w2, b2, batch, embed, ncls):
    blk = 512

    def body(p_ref, w1_ref, b1_ref, w2_ref, b2_ref, o_ref):
        h = jnp.dot(p_ref[...], w1_ref[...], preferred_element_type=jnp.float32)
        h = jnp.maximum(h + b1_ref[...], 0.0)
        o = jnp.dot(h, w2_ref[...], preferred_element_type=jnp.float32)
        o_ref[...] = o + b2_ref[...]

    return pl.pallas_call(
        body,
        grid=(batch // blk,),
        in_specs=[
            pl.BlockSpec((blk, embed), lambda i: (i, 0)),
            pl.BlockSpec((embed, embed), lambda i: (0, 0)),
            pl.BlockSpec((1, embed), lambda i: (0, 0)),
            pl.BlockSpec((embed, ncls), lambda i: (0, 0)),
            pl.BlockSpec((1, ncls), lambda i: (0, 0)),
        ],
        out_specs=pl.BlockSpec((blk, ncls), lambda i: (i, 0)),
        out_shape=jax.ShapeDtypeStruct((batch, ncls), jnp.float32),
    )(pooled, w1, b1, w2, b2)


def kernel(x, table, W1, b1, W2, b2):
    batch, hist = x.shape
    vocab, embed = table.shape
    ncls = W2.shape[1]

    packed = _tc_pack(table.T, vocab, embed)
    packed_flat = packed.reshape(-1, embed)

    bpw = batch // _NW
    xw = x.astype(jnp.int32).reshape(_NW, bpw * hist)

    pooled_sum = _sc_pool(xw, packed_flat, batch, hist, embed)

    w1s = W1 * (1.0 / hist)
    out = _tc_mlp(
        pooled_sum,
        w1s,
        b1.reshape(1, embed),
        W2,
        b2.reshape(1, ncls),
        batch,
        embed,
        ncls,
    )
    return out


# speedup vs baseline: 2.3610x; 1.1648x over previous
import functools

import jax
import jax.numpy as jnp
from jax import lax
from jax.experimental import pallas as pl
from jax.experimental.pallas import tpu as pltpu
from jax.experimental.pallas import tpu_sc as plsc

_NC = 2
_NS = 16
_NW = _NC * _NS
_L = 16
_NBUF = 6

_PACK_BLK = 4096
_PACK_B = 124 * _PACK_BLK


def _tc_pack(table_t, vocab, embed):
    nblk = _PACK_B // _PACK_BLK

    last_blk = (vocab - 1) // _PACK_BLK

    def body(lo_ref, hi_ref, eye_ref, o_ref):
        dn = (((0,), (0,)), ((), ()))
        o_ref[:, 0:embed] = lax.dot_general(
            lo_ref[...], eye_ref[...], dn,
            preferred_element_type=jnp.float32,
        )
        o_ref[:, embed:2 * embed] = lax.dot_general(
            hi_ref[...], eye_ref[...], dn,
            preferred_element_type=jnp.float32,
        )

    return pl.pallas_call(
        body,
        grid=(nblk,),
        compiler_params=pltpu.CompilerParams(
            fuse_transposed_lhs_in_matmul=True
        ),
        in_specs=[
            pl.BlockSpec((embed, _PACK_BLK), lambda i: (0, i)),
            pl.BlockSpec(
                (embed, _PACK_BLK),
                lambda i: (0, jnp.minimum(i + nblk, last_blk)),
            ),
            pl.BlockSpec((embed, embed), lambda i: (0, 0)),
        ],
        out_specs=pl.BlockSpec((_PACK_BLK, 2 * embed), lambda i: (i, 0)),
        out_shape=jax.ShapeDtypeStruct((_PACK_B, 2 * embed), jnp.float32),
    )(table_t, table_t, jnp.eye(embed, dtype=jnp.float32))


def _sc_pool(x_by_worker, packed_flat, batch, hist, embed):
    bpw = batch // _NW
    assert bpw % 8 == 0 and bpw <= 128
    nvec = embed // _L
    hist_pad = hist + (-hist) % _NBUF
    mesh = plsc.VectorSubcoreMesh(core_axis_name="c", subcore_axis_name="s")

    @functools.partial(
        pl.kernel,
        mesh=mesh,
        compiler_params=pltpu.CompilerParams(
            use_tc_tiling_on_sc=False, needs_layout_passes=False
        ),
        out_type=jax.ShapeDtypeStruct((batch, embed), jnp.float32),
        scratch_types=(
            [pltpu.VMEM((bpw * hist,), jnp.int32)]
            + [pltpu.VMEM((hist, bpw), jnp.int32)]
            + [pltpu.VMEM((bpw, embed), jnp.float32) for _ in range(_NBUF)]
            + [pltpu.VMEM((bpw, embed), jnp.float32)]
            + [pltpu.SemaphoreType.DMA for _ in range(_NBUF)]
        ),
    )
    def pool(x_hbm, tab_hbm, out_hbm, idx_raw, idx_v, *rest):
        bufs = rest[:_NBUF]
        acc = rest[_NBUF]
        sems = rest[_NBUF + 1:]
        wid = lax.axis_index("s") * _NC + lax.axis_index("c")
        base = wid * bpw

        pltpu.sync_copy(x_hbm.at[wid], idx_raw)

        lanes = jnp.arange(_L, dtype=jnp.int32)
        def tbody(j, c):
            for g in range(bpw // _L):
                offs = (g * _L + lanes) * hist + j
                v = plsc.load_gather(idx_raw, [offs])
                ge = (v >= _PACK_B).astype(jnp.int32)
                idx_v[j, pl.ds(g * _L, _L)] = 2 * v - ge * (2 * _PACK_B - 1)
            return c
        lax.fori_loop(0, hist, tbody, 0, unroll=2)

        def gcopy(j, b):
            return pltpu.make_async_copy(
                tab_hbm.at[idx_v.at[j]], bufs[b], sems[b]
            )

        zeros = jnp.zeros((_L,), jnp.float32)
        for j in range(_NBUF):
            gcopy(j, j).start()
        def zbody(r, c):
            for p in range(nvec):
                acc[r, pl.ds(p * _L, _L)] = zeros
            return c
        lax.fori_loop(0, bpw, zbody, 0, unroll=8)

        def accum(buf):
            def body(r, c):
                for p in range(nvec):
                    sl = pl.ds(p * _L, _L)
                    acc[r, sl] = acc[r, sl] + buf[r, sl]
                return c
            lax.fori_loop(0, bpw, body, 0, unroll=4)

        for j in range(hist):
            b = j % _NBUF
            gcopy(j, b).wait()
            accum(bufs[b])
            nj = j + _NBUF
            if nj < hist:
                gcopy(nj, b).start()

        pltpu.sync_copy(acc, out_hbm.at[pl.ds(base, bpw)])

    return pool(x_by_worker, packed_flat)


def _tc_mlp(pooled, w1, b1, w2, b2, batch, embed, ncls):
    blk = 512

    def body(p_ref, w1_ref, b1_ref, w2_ref, b2_ref, o_ref):
        h = jnp.dot(p_ref[...], w1_ref[...], preferred_element_type=jnp.float32)
        h = jnp.maximum(h + b1_ref[...], 0.0)
        o = jnp.dot(h, w2_ref[...], preferred_element_type=jnp.float32)
        o_ref[...] = o + b2_ref[...]

    return pl.pallas_call(
        body,
        grid=(batch // blk,),
        in_specs=[
            pl.BlockSpec((blk, embed), lambda i: (i, 0)),
            pl.BlockSpec((embed, embed), lambda i: (0, 0)),
            pl.BlockSpec((1, embed), lambda i: (0, 0)),
            pl.BlockSpec((embed, ncls), lambda i: (0, 0)),
            pl.BlockSpec((1, ncls), lambda i: (0, 0)),
        ],
        out_specs=pl.BlockSpec((blk, ncls), lambda i: (i, 0)),
        out_shape=jax.ShapeDtypeStruct((batch, ncls), jnp.float32),
    )(pooled, w1, b1, w2, b2)


def kernel(x, table, W1, b1, W2, b2):
    batch, hist = x.shape
    vocab, embed = table.shape
    ncls = W2.shape[1]

    packed = _tc_pack(table.T, vocab, embed)
    packed_flat = packed.reshape(-1, embed)

    bpw = batch // _NW
    xw = x.astype(jnp.int32).reshape(_NW, bpw * hist)

    pooled_sum = _sc_pool(xw, packed_flat, batch, hist, embed)

    w1s = W1 * (1.0 / hist)
    out = _tc_mlp(
        pooled_sum,
        w1s,
        b1.reshape(1, embed),
        W2,
        b2.reshape(1, ncls),
        batch,
        embed,
        ncls,
    )
    return out


# speedup vs baseline: 2.5795x; 1.0925x over previous
import functools

import jax
import jax.numpy as jnp
from jax import lax
from jax.experimental import pallas as pl
from jax.experimental.pallas import tpu as pltpu
from jax.experimental.pallas import tpu_sc as plsc

_NC = 2
_NS = 16
_NW = _NC * _NS
_L = 16
_NBUF = 6

_PACK_BLK = 4096
_PACK_B = 124 * _PACK_BLK


def _tc_pack(table_t, vocab, embed):
    nblk = _PACK_B // _PACK_BLK

    last_blk = (vocab - 1) // _PACK_BLK

    def body(lo_ref, hi_ref, eye_ref, o_ref):
        dn = (((0,), (0,)), ((), ()))
        o_ref[:, 0:embed] = lax.dot_general(
            lo_ref[...], eye_ref[...], dn,
            preferred_element_type=jnp.float32,
        )
        o_ref[:, embed:2 * embed] = lax.dot_general(
            hi_ref[...], eye_ref[...], dn,
            preferred_element_type=jnp.float32,
        )

    return pl.pallas_call(
        body,
        grid=(nblk,),
        compiler_params=pltpu.CompilerParams(
            fuse_transposed_lhs_in_matmul=True
        ),
        in_specs=[
            pl.BlockSpec((embed, _PACK_BLK), lambda i: (0, i)),
            pl.BlockSpec(
                (embed, _PACK_BLK),
                lambda i: (0, jnp.minimum(i + nblk, last_blk)),
            ),
            pl.BlockSpec((embed, embed), lambda i: (0, 0)),
        ],
        out_specs=pl.BlockSpec((_PACK_BLK, 2 * embed), lambda i: (i, 0)),
        out_shape=jax.ShapeDtypeStruct((_PACK_B, 2 * embed), jnp.float32),
    )(table_t, table_t, jnp.eye(embed, dtype=jnp.float32))


def _sc_pool(x_by_worker, packed_flat, batch, hist, embed):
    bpw = batch // _NW
    assert bpw % 8 == 0 and bpw <= 128
    nvec = embed // _L
    hist_pad = hist + (-hist) % _NBUF
    mesh = plsc.VectorSubcoreMesh(core_axis_name="c", subcore_axis_name="s")

    @functools.partial(
        pl.kernel,
        mesh=mesh,
        compiler_params=pltpu.CompilerParams(
            use_tc_tiling_on_sc=False, needs_layout_passes=False
        ),
        out_type=jax.ShapeDtypeStruct((batch, embed), jnp.float32),
        scratch_types=(
            [pltpu.VMEM((bpw * hist,), jnp.int32)]
            + [pltpu.VMEM((hist, bpw), jnp.int32)]
            + [pltpu.VMEM((bpw, embed), jnp.float32) for _ in range(_NBUF)]
            + [pltpu.VMEM((bpw, embed), jnp.float32)]
            + [pltpu.SemaphoreType.DMA for _ in range(_NBUF)]
        ),
    )
    def pool(x_hbm, tab_hbm, out_hbm, idx_raw, idx_v, *rest):
        bufs = rest[:_NBUF]
        acc = rest[_NBUF]
        sems = rest[_NBUF + 1:]
        wid = lax.axis_index("s") * _NC + lax.axis_index("c")
        base = wid * bpw

        pltpu.sync_copy(x_hbm.at[wid], idx_raw)

        lanes = jnp.arange(_L, dtype=jnp.int32)
        def tbody(j, c):
            for g in range(bpw // _L):
                offs = (g * _L + lanes) * hist + j
                v = plsc.load_gather(idx_raw, [offs])
                ge = (v >= _PACK_B).astype(jnp.int32)
                idx_v[j, pl.ds(g * _L, _L)] = 2 * v - ge * (2 * _PACK_B - 1)
            return c
        lax.fori_loop(0, hist, tbody, 0, unroll=2)

        def gcopy(j, b):
            return pltpu.make_async_copy(
                tab_hbm.at[idx_v.at[j]], bufs[b], sems[b]
            )

        zeros = jnp.zeros((_L,), jnp.float32)
        for j in range(_NBUF):
            gcopy(j, j).start()
        def zbody(r, c):
            for p in range(nvec):
                acc[r, pl.ds(p * _L, _L)] = zeros
            return c
        lax.fori_loop(0, bpw, zbody, 0, unroll=8)

        def accum2(ba, bb):
            def body(r, c):
                for p in range(nvec):
                    sl = pl.ds(p * _L, _L)
                    acc[r, sl] = acc[r, sl] + (ba[r, sl] + bb[r, sl])
                return c
            lax.fori_loop(0, bpw, body, 0, unroll=4)

        def accum(buf):
            def body(r, c):
                for p in range(nvec):
                    sl = pl.ds(p * _L, _L)
                    acc[r, sl] = acc[r, sl] + buf[r, sl]
                return c
            lax.fori_loop(0, bpw, body, 0, unroll=4)

        for j in range(0, hist - hist % 2, 2):
            b0, b1 = j % _NBUF, (j + 1) % _NBUF
            gcopy(j, b0).wait()
            gcopy(j + 1, b1).wait()
            accum2(bufs[b0], bufs[b1])
            for nj in (j + _NBUF, j + 1 + _NBUF):
                if nj < hist:
                    gcopy(nj, nj % _NBUF).start()
        for j in range(hist - hist % 2, hist):
            b = j % _NBUF
            gcopy(j, b).wait()
            accum(bufs[b])

        pltpu.sync_copy(acc, out_hbm.at[pl.ds(base, bpw)])

    return pool(x_by_worker, packed_flat)


def _tc_mlp(pooled, w1, b1, w2, b2, batch, embed, ncls):
    blk = 512

    def body(p_ref, w1_ref, b1_ref, w2_ref, b2_ref, o_ref):
        h = jnp.dot(p_ref[...], w1_ref[...], preferred_element_type=jnp.float32)
        h = jnp.maximum(h + b1_ref[...], 0.0)
        o = jnp.dot(h, w2_ref[...], preferred_element_type=jnp.float32)
        o_ref[...] = o + b2_ref[...]

    return pl.pallas_call(
        body,
        grid=(batch // blk,),
        in_specs=[
            pl.BlockSpec((blk, embed), lambda i: (i, 0)),
            pl.BlockSpec((embed, embed), lambda i: (0, 0)),
            pl.BlockSpec((1, embed), lambda i: (0, 0)),
            pl.BlockSpec((embed, ncls), lambda i: (0, 0)),
            pl.BlockSpec((1, ncls), lambda i: (0, 0)),
        ],
        out_specs=pl.BlockSpec((blk, ncls), lambda i: (i, 0)),
        out_shape=jax.ShapeDtypeStruct((batch, ncls), jnp.float32),
    )(pooled, w1, b1, w2, b2)


def kernel(x, table, W1, b1, W2, b2):
    batch, hist = x.shape
    vocab, embed = table.shape
    ncls = W2.shape[1]

    packed = _tc_pack(table.T, vocab, embed)
    packed_flat = packed.reshape(-1, embed)

    bpw = batch // _NW
    xw = x.astype(jnp.int32).reshape(_NW, bpw * hist)

    pooled_sum = _sc_pool(xw, packed_flat, batch, hist, embed)

    w1s = W1 * (1.0 / hist)
    out = _tc_mlp(
        pooled_sum,
        w1s,
        b1.reshape(1, embed),
        W2,
        b2.reshape(1, ncls),
        batch,
        embed,
        ncls,
    )
    return out
